# baseline (speedup 1.0000x reference)
import functools

import jax
import jax.numpy as jnp
from jax import lax
from jax.experimental import pallas as pl
from jax.experimental.pallas import tpu as pltpu
from jax.experimental.pallas import tpu_sc as plsc

B, N, H, W, D = 128, 10, 30, 30, 64
NUM_COLORS = 11
HW = H * W
P = N * NUM_COLORS
CELLS = B * N * HW
PLANES = B * N
NW = 32
PPT = PLANES // NW
CHUNK = 120
FCOLS = 1024


def _mega_body(color_ref, row_ref, col_ref, ex_ref, role_ref, out_ref):
    n = pl.program_id(0)
    c = pl.program_id(1)
    fused = (color_ref[pl.ds(c, 1), :]
             + ex_ref[pl.ds(n // 2 + 1, 1), :]
             + role_ref[pl.ds(n % 2, 1), :])
    out_ref[0] = fused[:, None, :] + row_ref[...][:, None, :] + col_ref[...][None, :, :]


def _build_mega(color_table, row_table, col_table, example_table, role_table):
    return pl.pallas_call(
        _mega_body,
        grid=(N, NUM_COLORS),
        in_specs=[
            pl.BlockSpec((NUM_COLORS, D), lambda n, c: (0, 0)),
            pl.BlockSpec((H, D), lambda n, c: (0, 0)),
            pl.BlockSpec((W, D), lambda n, c: (0, 0)),
            pl.BlockSpec((NUM_COLORS, D), lambda n, c: (0, 0)),
            pl.BlockSpec((2, D), lambda n, c: (0, 0)),
        ],
        out_specs=pl.BlockSpec((1, H, W, D), lambda n, c: (n * NUM_COLORS + c, 0, 0, 0)),
        out_shape=jax.ShapeDtypeStruct((P, H, W, D), jnp.float32),
    )(color_table, row_table, col_table, example_table, role_table)


def _fidx_body(g_ref, out_ref):
    t = pl.program_id(0)
    n_l = (t * 8 + lax.broadcasted_iota(jnp.int32, (8, HW), 0)) % N
    hw = lax.broadcasted_iota(jnp.int32, (8, HW), 1)
    out_ref[:, pl.ds(0, HW)] = g_ref[...] * HW + n_l * (NUM_COLORS * HW) + hw
    out_ref[:, pl.ds(HW, FCOLS - HW)] = jnp.zeros((8, FCOLS - HW), jnp.int32)


def _build_fidx(grids2):
    return pl.pallas_call(
        _fidx_body,
        grid=(PLANES // 8,),
        in_specs=[pl.BlockSpec((8, HW), lambda t: (t, 0))],
        out_specs=pl.BlockSpec((8, FCOLS), lambda t: (t, 0)),
        out_shape=jax.ShapeDtypeStruct((PLANES, FCOLS), jnp.int32),
    )(grids2)


_MESH = plsc.VectorSubcoreMesh(core_axis_name="c", subcore_axis_name="s")


@functools.partial(
    pl.kernel,
    mesh=_MESH,
    compiler_params=pltpu.CompilerParams(use_tc_tiling_on_sc=False),
    out_type=jax.ShapeDtypeStruct((B, N, H, W, D), jnp.float32),
    scratch_types=[
        pltpu.VMEM((2, FCOLS), jnp.int32),
        pltpu.VMEM((2, HW + 4, D), jnp.float32),
        pltpu.SemaphoreType.DMA,
        pltpu.SemaphoreType.DMA,
        pltpu.SemaphoreType.DMA,
        pltpu.SemaphoreType.DMA,
    ],
)
def _sc_gather(mega_hbm, fidx_hbm, out5_hbm, idx_v, rows_v, gs0, gs1, ss0, ss1):
    gsem = (gs0, gs1)
    ssem = (ss0, ss1)
    wid = lax.axis_index("s") * 2 + lax.axis_index("c")
    plane0 = wid * PPT

    def load_and_fire(i, b):
        plane = plane0 + i
        pltpu.sync_copy(fidx_hbm.at[plane], idx_v.at[b])
        for j in range(8):
            cnt = CHUNK if j < 7 else HW + 4 - 7 * CHUNK
            pltpu.async_copy(
                mega_hbm.at[idx_v.at[b, pl.ds(j * CHUNK, cnt)]],
                rows_v.at[b, pl.ds(j * CHUNK, cnt)],
                gsem[b],
            )

    def drain_gathers(b):
        pltpu.make_async_copy(mega_hbm.at[pl.ds(0, HW)], rows_v.at[b],
                              gsem[b]).wait()

    def fire_scatter(i, b):
        plane = plane0 + i
        bb = plane // N
        nn = plane % N
        for hh in range(H):
            pltpu.async_copy(rows_v.at[b, pl.ds(hh * W, W)],
                             out5_hbm.at[bb, nn, hh], ssem[b])

    def wait_scatter(b):
        pltpu.make_async_copy(mega_hbm.at[pl.ds(0, HW)],
                              rows_v.at[b, pl.ds(0, HW)], ssem[b]).wait()

    load_and_fire(0, 0)
    drain_gathers(0)
    fire_scatter(0, 0)
    load_and_fire(1, 1)

    def body(k, carry):
        for i_off, b in ((1, 1), (2, 0)):
            i = 2 * k + i_off
            drain_gathers(b)
            fire_scatter(i, b)
            wait_scatter(1 - b)
            load_and_fire(i + 1, 1 - b)
        return carry

    lax.fori_loop(0, (PPT - 2) // 2, body, 0)

    drain_gathers(1)
    fire_scatter(PPT - 1, 1)
    wait_scatter(0)
    wait_scatter(1)


def kernel(grids, color_table, row_table, col_table, example_table, role_table):
    grids = grids.astype(jnp.int32)
    mega = _build_mega(color_table, row_table, col_table, example_table, role_table)
    mega = mega.reshape(P * HW, D)
    fidx = _build_fidx(grids.reshape(PLANES, HW))
    return _sc_gather(mega, fidx)


# speedup vs baseline: 1.2845x; 1.2845x over previous
import functools

import jax
import jax.numpy as jnp
from jax import lax
from jax.experimental import pallas as pl
from jax.experimental.pallas import tpu as pltpu
from jax.experimental.pallas import tpu_sc as plsc

B, N, H, W, D = 128, 10, 30, 30, 64
NUM_COLORS = 11
HW = H * W
P = N * NUM_COLORS
CELLS = B * N * HW
NW = 32
CPT = CELLS // NW
CHUNK = 120
KCH = 5
SUPER = CHUNK * KCH
ITERS = CPT // SUPER
NBUF = 3


def _mega_body(color_ref, row_ref, col_ref, ex_ref, role_ref, out_ref):
    n = pl.program_id(0)
    exro = ex_ref[pl.ds(n // 2 + 1, 1), :] + role_ref[pl.ds(n % 2, 1), :]
    out_ref[...] = (color_ref[...][:, None, None, :]
                    + row_ref[...][None, :, None, :]
                    + col_ref[...][None, None, :, :]
                    + exro[:, None, None, :])


def _build_mega(color_table, row_table, col_table, example_table, role_table):
    return pl.pallas_call(
        _mega_body,
        grid=(N,),
        in_specs=[
            pl.BlockSpec((NUM_COLORS, D), lambda n: (0, 0)),
            pl.BlockSpec((H, D), lambda n: (0, 0)),
            pl.BlockSpec((W, D), lambda n: (0, 0)),
            pl.BlockSpec((NUM_COLORS, D), lambda n: (0, 0)),
            pl.BlockSpec((2, D), lambda n: (0, 0)),
        ],
        out_specs=pl.BlockSpec((NUM_COLORS, H, W, D), lambda n: (n, 0, 0, 0)),
        out_shape=jax.ShapeDtypeStruct((P, H, W, D), jnp.float32),
    )(color_table, row_table, col_table, example_table, role_table)


_FB = 4


def _fidx_body(g_ref, out_ref):
    n_l = lax.broadcasted_iota(jnp.int32, (_FB, N, HW), 1)
    hw = lax.broadcasted_iota(jnp.int32, (_FB, N, HW), 2)
    out_ref[...] = g_ref[...] * HW + n_l * (NUM_COLORS * HW) + hw


def _build_fidx(grids3):
    return pl.pallas_call(
        _fidx_body,
        grid=(B // _FB,),
        in_specs=[pl.BlockSpec((_FB, N, HW), lambda b: (b, 0, 0))],
        out_specs=pl.BlockSpec((_FB, N, HW), lambda b: (b, 0, 0)),
        out_shape=jax.ShapeDtypeStruct((B, N, HW), jnp.int32),
    )(grids3)


_MESH = plsc.VectorSubcoreMesh(core_axis_name="c", subcore_axis_name="s")


@functools.partial(
    pl.kernel,
    mesh=_MESH,
    compiler_params=pltpu.CompilerParams(use_tc_tiling_on_sc=False),
    out_type=jax.ShapeDtypeStruct((CELLS, D), jnp.float32),
    scratch_types=[
        pltpu.VMEM((NBUF, SUPER), jnp.int32),
        pltpu.VMEM((NBUF, SUPER, D), jnp.float32),
        pltpu.SemaphoreType.DMA,
        pltpu.SemaphoreType.DMA,
        pltpu.SemaphoreType.DMA,
        pltpu.SemaphoreType.DMA,
        pltpu.SemaphoreType.DMA,
        pltpu.SemaphoreType.DMA,
    ],
)
def _sc_gather(mega_hbm, fidx_hbm, out_hbm, idx_v, rows_v,
               gs0, gs1, gs2, ss0, ss1, ss2):
    gsem = (gs0, gs1, gs2)
    ssem = (ss0, ss1, ss2)
    wid = lax.axis_index("s") * 2 + lax.axis_index("c")
    cell0 = wid * CPT

    def load_and_fire(i, b):
        base_cell = cell0 + i * SUPER
        pltpu.sync_copy(fidx_hbm.at[pl.ds(base_cell, SUPER)], idx_v.at[b])
        for j in range(KCH):
            pltpu.async_copy(
                mega_hbm.at[idx_v.at[b, pl.ds(j * CHUNK, CHUNK)]],
                rows_v.at[b, pl.ds(j * CHUNK, CHUNK)],
                gsem[b],
            )

    def drain_gathers(b):
        pltpu.make_async_copy(out_hbm.at[pl.ds(0, SUPER)], rows_v.at[b],
                              gsem[b]).wait()

    def fire_scatter(i, b):
        pltpu.async_copy(rows_v.at[b],
                         out_hbm.at[pl.ds(cell0 + i * SUPER, SUPER)], ssem[b])

    def wait_scatter(b):
        pltpu.make_async_copy(out_hbm.at[pl.ds(0, SUPER)], rows_v.at[b],
                              ssem[b]).wait()

    load_and_fire(0, 0)
    load_and_fire(1, 1)
    drain_gathers(0)
    fire_scatter(0, 0)
    load_and_fire(2, 2)

    def body(k, carry):
        for j in range(3):
            i = 1 + 3 * k + j
            b = (1 + j) % 3
            drain_gathers(b)
            fire_scatter(i, b)
            wait_scatter(j)
            load_and_fire(i + 2, j)
        return carry

    lax.fori_loop(0, (ITERS - 3) // 3, body, 0)

    for i in (ITERS - 2, ITERS - 1):
        b = i % 3
        drain_gathers(b)
        fire_scatter(i, b)
    for b in range(3):
        wait_scatter(b)


def kernel(grids, color_table, row_table, col_table, example_table, role_table):
    grids = grids.astype(jnp.int32)
    mega = _build_mega(color_table, row_table, col_table, example_table, role_table)
    mega = mega.reshape(P * HW, D)
    fidx = _build_fidx(grids.reshape(B, N, HW))
    fidx = fidx.reshape(CELLS)
    out = _sc_gather(mega, fidx)
    return out.reshape(B, N, H, W, D)
